# baseline (speedup 1.0000x reference)
import functools
import math

import jax
import jax.numpy as jnp
from jax.experimental import pallas as pl

_B, _N = 4, 256
_F = 64
_H = 64
_ENF = 16
_BI = 128
_NI = _N // _BI


def _sinusoidal(x, dim):
    half = dim // 2
    freqs = jnp.exp(-jnp.log(10000.0) * jnp.arange(half, dtype=jnp.float32) / half)
    args = x[..., None] * freqs
    return jnp.concatenate([jnp.sin(args), jnp.cos(args)], axis=-1)


def _embed_kernel(feat_ref, w_ref, b_ref, out_ref):
    out_ref[...] = (
        jnp.dot(feat_ref[...], w_ref[...], preferred_element_type=jnp.float32)
        + b_ref[...]
    )


def _layer_kernel(final,
                  xi_ref, xall_ref, xr_ref, bpi_ref, bpr_ref,
                  hi_ref, hall_ref, na_ref, feat_ref,
                  w1hi_ref, w1hj_ref, w1rad_ref, w1ea_ref, b1_ref,
                  w2_ref, b2_ref,
                  cw1_ref, cb1_ref, cw2t_ref, cb2_ref,
                  nw1h_ref, nw1a_ref, nw1n_ref, nb1_ref,
                  nw2_ref, nb2_ref,
                  wout_ref, bout_ref,
                  xout_ref, hout_ref):
    xi = xi_ref[0]
    xr = xr_ref[0]
    rad = ((xi[:, 0:1] - xr[0:1, :]) ** 2
           + (xi[:, 1:2] - xr[1:2, :]) ** 2
           + (xi[:, 2:3] - xr[2:3, :]) ** 2)
    norm = jnp.sqrt(rad + 1e-8)

    bpi = bpi_ref[0]
    bpr = bpr_ref[0]
    rad0 = ((bpi[:, 0:1] - bpr[0:1, :]) ** 2
            + (bpi[:, 1:2] - bpr[1:2, :]) ** 2
            + (bpi[:, 2:3] - bpr[2:3, :]) ** 2)
    d0 = jnp.sqrt(rad0 + 1e-8)
    half = _ENF // 2
    k = jax.lax.broadcasted_iota(jnp.int32, (1, 1, half), 2).astype(jnp.float32)
    freqs = jnp.exp((-math.log(10000.0) / half) * k)
    args = d0[:, :, None] * freqs
    ea = jnp.concatenate([jnp.sin(args), jnp.cos(args)], axis=2)
    ea_c = jnp.dot(ea.reshape(_BI * _N, _ENF), w1ea_ref[...],
                   preferred_element_type=jnp.float32)

    hi = hi_ref[0]
    hall = hall_ref[0]
    a_i = jnp.dot(hi, w1hi_ref[...], preferred_element_type=jnp.float32)
    a_j = jnp.dot(hall, w1hj_ref[...], preferred_element_type=jnp.float32)

    m = (a_i[:, None, :] + a_j[None, :, :]
         + rad[:, :, None] * w1rad_ref[...].reshape(1, 1, _H)
         + ea_c.reshape(_BI, _N, _H)
         + b1_ref[...].reshape(1, 1, _H))
    m = jax.nn.silu(m)
    m = jax.nn.silu(
        jnp.dot(m.reshape(_BI * _N, _H), w2_ref[...],
                preferred_element_type=jnp.float32) + b2_ref[...])

    i_idx = jax.lax.broadcasted_iota(jnp.int32, (_BI, _N), 0) + pl.program_id(1) * _BI
    j_idx = jax.lax.broadcasted_iota(jnp.int32, (_BI, _N), 1)
    adj = jnp.where(i_idx == j_idx, 0.0, 1.0)
    m3 = m.reshape(_BI, _N, _H) * adj[:, :, None]
    agg = jnp.sum(m3, axis=1)

    cwa = jax.nn.silu(
        jnp.dot(m3.reshape(_BI * _N, _H), cw1_ref[...],
                preferred_element_type=jnp.float32) + cb1_ref[...])
    cw = (jnp.sum(cwa.reshape(_BI, _N, _H) * cw2t_ref[...].reshape(1, 1, _H),
                  axis=2)
          + cb2_ref[...])
    wc = cw * adj / (norm + 1.0)
    rowsum = jnp.sum(wc, axis=1, keepdims=True)
    sj = jnp.dot(wc, xall_ref[0], preferred_element_type=jnp.float32)
    x_new = xi + (xi * rowsum - sj) * (1.0 / (_N - 1))

    na = na_ref[0]
    out = jax.nn.silu(
        jnp.dot(hi, nw1h_ref[...], preferred_element_type=jnp.float32)
        + jnp.dot(agg, nw1a_ref[...], preferred_element_type=jnp.float32)
        + jnp.dot(na, nw1n_ref[...], preferred_element_type=jnp.float32)
        + nb1_ref[...])
    out = jnp.dot(out, nw2_ref[...], preferred_element_type=jnp.float32) + nb2_ref[...]
    h_new = hi + out

    if final:
        xout_ref[0] = x_new - bpi
        hout_ref[0] = (jnp.dot(h_new, wout_ref[...],
                               preferred_element_type=jnp.float32)
                       + bout_ref[...]) - feat_ref[0]
    else:
        xout_ref[0] = x_new
        hout_ref[0] = h_new


def _egnn_layer(x, bb_pos, bbT, h, node_attr, features, lp, wout, bout, final):
    xT = jnp.swapaxes(x, 1, 2)
    ew1 = lp["edge_w1"]
    w1hi, w1hj = ew1[0:_H], ew1[_H:2 * _H]
    w1rad, w1ea = ew1[2 * _H:2 * _H + 1], ew1[2 * _H + 1:]
    nw1 = lp["node_w1"]
    nw1h, nw1a, nw1n = nw1[0:_H], nw1[_H:2 * _H], nw1[2 * _H:]

    def r2(v):
        return v.reshape(1, -1)

    blk_i3 = pl.BlockSpec((1, _BI, 3), lambda b, i: (b, i, 0))
    blk_n3 = pl.BlockSpec((1, _N, 3), lambda b, i: (b, 0, 0))
    blk_3n = pl.BlockSpec((1, 3, _N), lambda b, i: (b, 0, 0))
    blk_ih = pl.BlockSpec((1, _BI, _H), lambda b, i: (b, i, 0))
    blk_nh = pl.BlockSpec((1, _N, _H), lambda b, i: (b, 0, 0))

    def wspec(a):
        return pl.BlockSpec(a.shape, lambda b, i: tuple(0 for _ in a.shape))

    weights = [w1hi, w1hj, r2(w1rad), w1ea, r2(lp["edge_b1"]),
               lp["edge_w2"], r2(lp["edge_b2"]),
               lp["coord_w1"], r2(lp["coord_b1"]),
               lp["coord_w2"].reshape(1, _H), lp["coord_b2"].reshape(1, 1),
               nw1h, nw1a, nw1n, r2(lp["node_b1"]),
               lp["node_w2"], r2(lp["node_b2"]),
               wout, r2(bout)]

    return pl.pallas_call(
        functools.partial(_layer_kernel, final),
        grid=(_B, _NI),
        in_specs=[blk_i3, blk_n3, blk_3n, blk_i3, blk_3n,
                  blk_ih, blk_nh, blk_ih, blk_ih]
                 + [wspec(w) for w in weights],
        out_specs=[blk_i3, blk_ih],
        out_shape=[jax.ShapeDtypeStruct((_B, _N, 3), jnp.float32),
                   jax.ShapeDtypeStruct((_B, _N, _H), jnp.float32)],
    )(x, x, xT, bb_pos, bbT, h, h, node_attr, features, *weights)


def kernel(coordinates, features, idx, params):
    bb_pos = coordinates.astype(jnp.float32)
    bb_feat = features.astype(jnp.float32)
    bbT = jnp.swapaxes(bb_pos, 1, 2)

    pos_ids = jnp.arange(_N, dtype=jnp.float32)
    embed_N = _sinusoidal(pos_ids, _H)
    embed_T = _sinusoidal(idx.astype(jnp.float32), _H)
    node_attr = (embed_N[None, :, :] + embed_T[:, None, :]).astype(jnp.float32)

    win, bin_ = params["emb_in"]
    h0 = pl.pallas_call(
        _embed_kernel,
        out_shape=jax.ShapeDtypeStruct((_B * _N, _H), jnp.float32),
    )(bb_feat.reshape(_B * _N, _F), win, bin_.reshape(1, _H))
    h = h0.reshape(_B, _N, _H)

    wout, bout = params["emb_out"]
    x = bb_pos
    n_layers = len(params["layers"])
    for li, lp in enumerate(params["layers"]):
        final = li == n_layers - 1
        x, h = _egnn_layer(x, bb_pos, bbT, h, node_attr, bb_feat, lp,
                           wout, bout, final)

    return (x, h)
